# Initial kernel scaffold; baseline (speedup 1.0000x reference)
import jax
import jax.numpy as jnp
from jax.experimental import pallas as pl


def kernel(x, edge_index, W1, b1, W2, b2, Wfc, bfc):
    raise NotImplementedError("write your pallas kernel here")


# speedup vs baseline: 8.8102x; 8.8102x over previous
import functools

import jax
import jax.numpy as jnp
from jax import lax
from jax.experimental import pallas as pl
from jax.experimental.pallas import tpu as pltpu
from jax.experimental.pallas import tpu_sc as plsc

NN = 10000
NE = 320000
D = 128
NC, NS = 2, 16
NW = NC * NS
CH = 128
NCHUNK = 80
EPW = NCHUNK * CH
NEP = NW * EPW
NNP = 10112
RPT = NNP // NS
DEGW = 128
TRASH = NN

_MESH = plsc.VectorSubcoreMesh(core_axis_name="c", subcore_axis_name="s")


def _deg_body(dst_hbm, ones_hbm, zeros_hbm, out_hbm, idx_v, ones_v, acc_sh):
    c = lax.axis_index("c")
    s = lax.axis_index("s")
    w = c * NS + s
    pltpu.sync_copy(zeros_hbm, acc_sh.at[pl.ds(s * RPT, RPT)])
    pltpu.sync_copy(ones_hbm, ones_v)
    pltpu.sync_copy(dst_hbm.at[w], idx_v)
    plsc.subcore_barrier()

    def body(j, carry):
        pltpu.sync_copy(ones_v, acc_sh.at[idx_v.at[j]], add=True)
        return carry

    lax.fori_loop(0, NCHUNK, body, 0)
    plsc.subcore_barrier()
    pltpu.sync_copy(
        acc_sh.at[pl.ds(s * RPT, RPT)],
        out_hbm.at[pl.ds(c * NNP + s * RPT, RPT)],
    )


_deg_call = pl.kernel(
    _deg_body,
    out_type=jax.ShapeDtypeStruct((NC * NNP, DEGW), jnp.float32),
    mesh=_MESH,
    scratch_types=[
        pltpu.VMEM((NCHUNK, CH), jnp.int32),
        pltpu.VMEM((CH, DEGW), jnp.float32),
        pltpu.VMEM_SHARED((NNP, DEGW), jnp.float32),
    ],
)


def _scat_body(g_hbm, src_hbm, dst_hbm, zeros_hbm, out_hbm,
               src_v, dst_v, rows0, sem0, acc_sh):
    c = lax.axis_index("c")
    s = lax.axis_index("s")
    w = c * NS + s
    pltpu.sync_copy(zeros_hbm, acc_sh.at[pl.ds(s * RPT, RPT)])
    pltpu.sync_copy(src_hbm.at[w], src_v)
    pltpu.sync_copy(dst_hbm.at[w], dst_v)
    plsc.subcore_barrier()

    def body(j, carry):
        pltpu.async_copy(g_hbm.at[src_v.at[j]], rows0, sem0).wait()
        pltpu.sync_copy(rows0, acc_sh.at[dst_v.at[j]], add=True)
        return carry

    lax.fori_loop(0, NCHUNK, body, 0)
    plsc.subcore_barrier()
    pltpu.sync_copy(
        acc_sh.at[pl.ds(s * RPT, RPT)],
        out_hbm.at[pl.ds(c * NNP + s * RPT, RPT)],
    )


_scatter_call = pl.kernel(
    _scat_body,
    out_type=jax.ShapeDtypeStruct((NC * NNP, D), jnp.float32),
    mesh=_MESH,
    scratch_types=[
        pltpu.VMEM((NCHUNK, CH), jnp.int32),
        pltpu.VMEM((NCHUNK, CH), jnp.int32),
        pltpu.VMEM((CH, D), jnp.float32),
        pltpu.SemaphoreType.DMA,
        pltpu.VMEM_SHARED((NNP, D), jnp.float32),
    ],
)


RB = 1000


def _tcb_body(degp_ref, x_ref, w_ref, dinv_ref, h_ref, g_ref):
    deg = degp_ref[0] + degp_ref[1] + 1.0
    dinv = lax.rsqrt(deg)
    h = jnp.dot(x_ref[...], w_ref[...], preferred_element_type=jnp.float32)
    dinv_ref[...] = dinv
    h_ref[...] = h
    g_ref[...] = dinv * h


def _tcb_call(degp, x, W1):
    return pl.pallas_call(
        _tcb_body,
        grid=(NN // RB,),
        in_specs=[
            pl.BlockSpec((NC, RB, 1), lambda i: (0, i, 0)),
            pl.BlockSpec((RB, D), lambda i: (i, 0)),
            pl.BlockSpec((D, D), lambda i: (0, 0)),
        ],
        out_specs=[
            pl.BlockSpec((RB, 1), lambda i: (i, 0)),
            pl.BlockSpec((RB, D), lambda i: (i, 0)),
            pl.BlockSpec((RB, D), lambda i: (i, 0)),
        ],
        out_shape=[
            jax.ShapeDtypeStruct((NN, 1), jnp.float32),
            jax.ShapeDtypeStruct((NN, D), jnp.float32),
            jax.ShapeDtypeStruct((NN, D), jnp.float32),
        ],
    )(degp, x, W1)


def _tcd_body(acc_ref, dinv_ref, h_ref, bpre_ref, w_ref, bpost_ref,
              h2_ref, g2_ref):
    dinv = dinv_ref[...]
    z = dinv * (acc_ref[0] + acc_ref[1] + dinv * h_ref[...]) + bpre_ref[...]
    z = jnp.maximum(z, 0.0)
    h2 = (jnp.dot(z, w_ref[...], preferred_element_type=jnp.float32)
          + bpost_ref[...])
    h2_ref[...] = h2
    g2_ref[...] = dinv * h2


def _tcd_call(acc, dinv, h, b_pre, W, b_post):
    return pl.pallas_call(
        _tcd_body,
        grid=(NN // RB,),
        in_specs=[
            pl.BlockSpec((NC, RB, D), lambda i: (0, i, 0)),
            pl.BlockSpec((RB, 1), lambda i: (i, 0)),
            pl.BlockSpec((RB, D), lambda i: (i, 0)),
            pl.BlockSpec((1, D), lambda i: (0, 0)),
            pl.BlockSpec((D, D), lambda i: (0, 0)),
            pl.BlockSpec((1, D), lambda i: (0, 0)),
        ],
        out_specs=[
            pl.BlockSpec((RB, D), lambda i: (i, 0)),
            pl.BlockSpec((RB, D), lambda i: (i, 0)),
        ],
        out_shape=[
            jax.ShapeDtypeStruct((NN, D), jnp.float32),
            jax.ShapeDtypeStruct((NN, D), jnp.float32),
        ],
    )(acc, dinv, h, b_pre, W, b_post)


@jax.jit
def kernel(x, edge_index, W1, b1, W2, b2, Wfc, bfc):
    ncls = Wfc.shape[1]
    ei = edge_index.astype(jnp.int32)
    npad = NEP - NE
    src = jnp.concatenate(
        [ei[0], jnp.zeros((npad,), jnp.int32)]).reshape(NW, NCHUNK, CH)
    dst = jnp.concatenate(
        [ei[1], jnp.full((npad,), TRASH, jnp.int32)]).reshape(NW, NCHUNK, CH)
    ones8 = jnp.ones((CH, DEGW), jnp.float32)
    zeros8 = jnp.zeros((RPT, DEGW), jnp.float32)
    zerosd = jnp.zeros((RPT, D), jnp.float32)

    degp = _deg_call(dst, ones8, zeros8)
    degp = degp[:, :1].reshape(NC, NNP, 1)[:, :NN]
    dinv, h1, g1 = _tcb_call(degp, x, W1)

    wfc_pad = jnp.pad(Wfc, ((0, 0), (0, D - ncls)))
    bfc_pad = jnp.pad(bfc, (0, D - ncls))
    b_pres = jnp.stack([b1, b2]).reshape(2, 1, D)
    ws = jnp.stack([W2, wfc_pad])
    b_posts = jnp.stack([jnp.zeros((D,), jnp.float32), bfc_pad]).reshape(
        2, 1, D)

    def step(carry, params):
        h, g = carry
        b_pre, w, b_post = params
        acc = _scatter_call(g, src, dst, zerosd).reshape(NC, NNP, D)[:, :NN]
        h2, g2 = _tcd_call(acc, dinv, h, b_pre, w, b_post)
        return (h2, g2), None

    (hf, _), _ = lax.scan(step, (h1, g1), (b_pres, ws, b_posts))
    return hf[:, :ncls]


# speedup vs baseline: 9.9344x; 1.1276x over previous
import functools

import jax
import jax.numpy as jnp
from jax import lax
from jax.experimental import pallas as pl
from jax.experimental.pallas import tpu as pltpu
from jax.experimental.pallas import tpu_sc as plsc

NN = 10000
NE = 320000
D = 128
NC, NS = 2, 16
NW = NC * NS
CH = 128
NCHUNK = 80
EPW = NCHUNK * CH
NEP = NW * EPW
NNP = 10112
RPT = NNP // NS
DEGW = 128
TRASH = NN

_MESH = plsc.VectorSubcoreMesh(core_axis_name="c", subcore_axis_name="s")


def _deg_body(dst_hbm, ones_hbm, zeros_hbm, out_hbm, idx_v, ones_v, acc_sh):
    c = lax.axis_index("c")
    s = lax.axis_index("s")
    w = c * NS + s
    pltpu.sync_copy(zeros_hbm, acc_sh.at[pl.ds(s * RPT, RPT)])
    pltpu.sync_copy(ones_hbm, ones_v)
    pltpu.sync_copy(dst_hbm.at[w], idx_v)
    plsc.subcore_barrier()

    def body(j, carry):
        pltpu.sync_copy(ones_v, acc_sh.at[idx_v.at[j]], add=True)
        return carry

    lax.fori_loop(0, NCHUNK, body, 0)
    plsc.subcore_barrier()
    pltpu.sync_copy(
        acc_sh.at[pl.ds(s * RPT, RPT)],
        out_hbm.at[pl.ds(c * NNP + s * RPT, RPT)],
    )


_deg_call = pl.kernel(
    _deg_body,
    out_type=jax.ShapeDtypeStruct((NC * NNP, DEGW), jnp.float32),
    mesh=_MESH,
    scratch_types=[
        pltpu.VMEM((NCHUNK, CH), jnp.int32),
        pltpu.VMEM((CH, DEGW), jnp.float32),
        pltpu.VMEM_SHARED((NNP, DEGW), jnp.float32),
    ],
)


NPHASE = 2
PH = NCHUNK // NPHASE


def _scat_body(g_hbm, src_hbm, dst_hbm, zeros_hbm, out_hbm,
               src_v, dst_v, rows0, rows1, sem0, sem1, acc_sh):
    c = lax.axis_index("c")
    s = lax.axis_index("s")
    w = c * NS + s
    pltpu.sync_copy(zeros_hbm, acc_sh.at[pl.ds(s * RPT, RPT)])
    plsc.subcore_barrier()

    for p in range(NPHASE):
        pltpu.sync_copy(src_hbm.at[w, pl.ds(p * PH, PH)], src_v)
        pltpu.sync_copy(dst_hbm.at[w, pl.ds(p * PH, PH)], dst_v)
        pltpu.async_copy(g_hbm.at[src_v.at[0]], rows0, sem0)

        def body(t, carry):
            j0 = 2 * t
            pltpu.async_copy(g_hbm.at[src_v.at[j0 + 1]], rows1, sem1)
            pltpu.make_async_copy(g_hbm.at[src_v.at[j0]], rows0, sem0).wait()
            pltpu.sync_copy(rows0, acc_sh.at[dst_v.at[j0]], add=True)

            @pl.when(j0 + 2 < PH)
            def _():
                pltpu.async_copy(g_hbm.at[src_v.at[j0 + 2]], rows0, sem0)

            pltpu.make_async_copy(
                g_hbm.at[src_v.at[j0 + 1]], rows1, sem1).wait()
            pltpu.sync_copy(rows1, acc_sh.at[dst_v.at[j0 + 1]], add=True)
            return carry

        lax.fori_loop(0, PH // 2, body, 0)
    plsc.subcore_barrier()
    pltpu.sync_copy(
        acc_sh.at[pl.ds(s * RPT, RPT)],
        out_hbm.at[pl.ds(c * NNP + s * RPT, RPT)],
    )


_scatter_call = pl.kernel(
    _scat_body,
    out_type=jax.ShapeDtypeStruct((NC * NNP, D), jnp.float32),
    mesh=_MESH,
    scratch_types=[
        pltpu.VMEM((PH, CH), jnp.int32),
        pltpu.VMEM((PH, CH), jnp.int32),
        pltpu.VMEM((CH, D), jnp.float32),
        pltpu.VMEM((CH, D), jnp.float32),
        pltpu.SemaphoreType.DMA,
        pltpu.SemaphoreType.DMA,
        pltpu.VMEM_SHARED((NNP, D), jnp.float32),
    ],
)


RB = 1000


def _tcb_body(degp_ref, x_ref, w_ref, dinv_ref, h_ref, g_ref):
    deg = degp_ref[0] + degp_ref[1] + 1.0
    dinv = lax.rsqrt(deg)
    h = jnp.dot(x_ref[...], w_ref[...], preferred_element_type=jnp.float32)
    dinv_ref[...] = dinv
    h_ref[...] = h
    g_ref[...] = dinv * h


def _tcb_call(degp, x, W1):
    return pl.pallas_call(
        _tcb_body,
        grid=(NN // RB,),
        in_specs=[
            pl.BlockSpec((NC, RB, 1), lambda i: (0, i, 0)),
            pl.BlockSpec((RB, D), lambda i: (i, 0)),
            pl.BlockSpec((D, D), lambda i: (0, 0)),
        ],
        out_specs=[
            pl.BlockSpec((RB, 1), lambda i: (i, 0)),
            pl.BlockSpec((RB, D), lambda i: (i, 0)),
            pl.BlockSpec((RB, D), lambda i: (i, 0)),
        ],
        out_shape=[
            jax.ShapeDtypeStruct((NN, 1), jnp.float32),
            jax.ShapeDtypeStruct((NN, D), jnp.float32),
            jax.ShapeDtypeStruct((NN, D), jnp.float32),
        ],
    )(degp, x, W1)


def _tcd_body(acc_ref, dinv_ref, h_ref, bpre_ref, w_ref, bpost_ref,
              h2_ref, g2_ref):
    dinv = dinv_ref[...]
    z = dinv * (acc_ref[0] + acc_ref[1] + dinv * h_ref[...]) + bpre_ref[...]
    z = jnp.maximum(z, 0.0)
    h2 = (jnp.dot(z, w_ref[...], preferred_element_type=jnp.float32)
          + bpost_ref[...])
    h2_ref[...] = h2
    g2_ref[...] = dinv * h2


def _tcd_call(acc, dinv, h, b_pre, W, b_post):
    return pl.pallas_call(
        _tcd_body,
        grid=(NN // RB,),
        in_specs=[
            pl.BlockSpec((NC, RB, D), lambda i: (0, i, 0)),
            pl.BlockSpec((RB, 1), lambda i: (i, 0)),
            pl.BlockSpec((RB, D), lambda i: (i, 0)),
            pl.BlockSpec((1, D), lambda i: (0, 0)),
            pl.BlockSpec((D, D), lambda i: (0, 0)),
            pl.BlockSpec((1, D), lambda i: (0, 0)),
        ],
        out_specs=[
            pl.BlockSpec((RB, D), lambda i: (i, 0)),
            pl.BlockSpec((RB, D), lambda i: (i, 0)),
        ],
        out_shape=[
            jax.ShapeDtypeStruct((NN, D), jnp.float32),
            jax.ShapeDtypeStruct((NN, D), jnp.float32),
        ],
    )(acc, dinv, h, b_pre, W, b_post)


@jax.jit
def kernel(x, edge_index, W1, b1, W2, b2, Wfc, bfc):
    ncls = Wfc.shape[1]
    ei = edge_index.astype(jnp.int32)
    npad = NEP - NE
    src = jnp.concatenate(
        [ei[0], jnp.zeros((npad,), jnp.int32)]).reshape(NW, NCHUNK, CH)
    dst = jnp.concatenate(
        [ei[1], jnp.full((npad,), TRASH, jnp.int32)]).reshape(NW, NCHUNK, CH)
    ones8 = jnp.ones((CH, DEGW), jnp.float32)
    zeros8 = jnp.zeros((RPT, DEGW), jnp.float32)
    zerosd = jnp.zeros((RPT, D), jnp.float32)

    degp = _deg_call(dst, ones8, zeros8)
    degp = degp[:, :1].reshape(NC, NNP, 1)[:, :NN]
    dinv, h1, g1 = _tcb_call(degp, x, W1)

    wfc_pad = jnp.pad(Wfc, ((0, 0), (0, D - ncls)))
    bfc_pad = jnp.pad(bfc, (0, D - ncls))
    b_pres = jnp.stack([b1, b2]).reshape(2, 1, D)
    ws = jnp.stack([W2, wfc_pad])
    b_posts = jnp.stack([jnp.zeros((D,), jnp.float32), bfc_pad]).reshape(
        2, 1, D)

    def step(carry, params):
        h, g = carry
        b_pre, w, b_post = params
        acc = _scatter_call(g, src, dst, zerosd).reshape(NC, NNP, D)[:, :NN]
        h2, g2 = _tcd_call(acc, dinv, h, b_pre, w, b_post)
        return (h2, g2), None

    (hf, _), _ = lax.scan(step, (h1, g1), (b_pres, ws, b_posts))
    return hf[:, :ncls]


# speedup vs baseline: 10.6509x; 1.0721x over previous
import functools

import jax
import jax.numpy as jnp
from jax import lax
from jax.experimental import pallas as pl
from jax.experimental.pallas import tpu as pltpu
from jax.experimental.pallas import tpu_sc as plsc

NN = 10000
NE = 320000
D = 128
NC, NS = 2, 16
NW = NC * NS
CH = 128
NCHUNK = 80
EPW = NCHUNK * CH
NEP = NW * EPW
NNP = 10112
RPT = NNP // NS
DEGW = 128
TRASH = NN

_MESH = plsc.VectorSubcoreMesh(core_axis_name="c", subcore_axis_name="s")


def _deg_body(dst_hbm, ones_hbm, zeros_hbm, out_hbm, idx_v, ones_v, acc_sh):
    c = lax.axis_index("c")
    s = lax.axis_index("s")
    w = c * NS + s
    pltpu.sync_copy(zeros_hbm, acc_sh.at[pl.ds(s * RPT, RPT)])
    pltpu.sync_copy(ones_hbm, ones_v)
    pltpu.sync_copy(dst_hbm.at[w], idx_v)
    plsc.subcore_barrier()

    def body(j, carry):
        pltpu.sync_copy(ones_v, acc_sh.at[idx_v.at[j]], add=True)
        return carry

    lax.fori_loop(0, NCHUNK, body, 0)
    plsc.subcore_barrier()
    pltpu.sync_copy(
        acc_sh.at[pl.ds(s * RPT, RPT)],
        out_hbm.at[pl.ds(c * NNP + s * RPT, RPT)],
    )


_deg_call = pl.kernel(
    _deg_body,
    out_type=jax.ShapeDtypeStruct((NC * NNP, DEGW), jnp.float32),
    mesh=_MESH,
    scratch_types=[
        pltpu.VMEM((NCHUNK, CH), jnp.int32),
        pltpu.VMEM((CH, DEGW), jnp.float32),
        pltpu.VMEM_SHARED((NNP, DEGW), jnp.float32),
    ],
)


NPHASE = 2
PH = NCHUNK // NPHASE


def _scat_body(g_hbm, src_hbm, dst_hbm, zeros_hbm, out_hbm,
               src_v, dst_v, rows0, rows1, sem0, sem1, acc_sh):
    c = lax.axis_index("c")
    s = lax.axis_index("s")
    w = c * NS + s
    pltpu.sync_copy(zeros_hbm, acc_sh.at[pl.ds(s * RPT, RPT)])
    plsc.subcore_barrier()

    for p in range(NPHASE):
        pltpu.sync_copy(src_hbm.at[w, pl.ds(p * PH, PH)], src_v)
        pltpu.sync_copy(dst_hbm.at[w, pl.ds(p * PH, PH)], dst_v)
        pltpu.async_copy(g_hbm.at[src_v.at[0]], rows0, sem0)

        def body(t, carry):
            j0 = 2 * t
            pltpu.async_copy(g_hbm.at[src_v.at[j0 + 1]], rows1, sem1)
            pltpu.make_async_copy(g_hbm.at[src_v.at[j0]], rows0, sem0).wait()
            pltpu.sync_copy(rows0, acc_sh.at[dst_v.at[j0]], add=True)

            @pl.when(j0 + 2 < PH)
            def _():
                pltpu.async_copy(g_hbm.at[src_v.at[j0 + 2]], rows0, sem0)

            pltpu.make_async_copy(
                g_hbm.at[src_v.at[j0 + 1]], rows1, sem1).wait()
            pltpu.sync_copy(rows1, acc_sh.at[dst_v.at[j0 + 1]], add=True)
            return carry

        lax.fori_loop(0, PH // 2, body, 0)
    plsc.subcore_barrier()
    pltpu.sync_copy(
        acc_sh.at[pl.ds(s * RPT, RPT)],
        out_hbm.at[pl.ds(c * NNP + s * RPT, RPT)],
    )


_scatter_call = pl.kernel(
    _scat_body,
    out_type=jax.ShapeDtypeStruct((NC * NNP, D), jnp.float32),
    mesh=_MESH,
    scratch_types=[
        pltpu.VMEM((PH, CH), jnp.int32),
        pltpu.VMEM((PH, CH), jnp.int32),
        pltpu.VMEM((CH, D), jnp.float32),
        pltpu.VMEM((CH, D), jnp.float32),
        pltpu.SemaphoreType.DMA,
        pltpu.SemaphoreType.DMA,
        pltpu.VMEM_SHARED((NNP, D), jnp.float32),
    ],
)


RB = 1000


def _tcb_body(degp_ref, x_ref, w_ref, dinv_ref, h_ref, g_ref):
    deg = degp_ref[0] + degp_ref[1] + 1.0
    dinv = lax.rsqrt(deg)
    h = jnp.dot(x_ref[...], w_ref[...], preferred_element_type=jnp.float32)
    dinv_ref[...] = dinv
    h_ref[...] = h
    g_ref[...] = dinv * h


def _tcb_call(degp, x, W1):
    return pl.pallas_call(
        _tcb_body,
        grid=(NN // RB,),
        in_specs=[
            pl.BlockSpec((NC, RB, 1), lambda i: (0, i, 0)),
            pl.BlockSpec((RB, D), lambda i: (i, 0)),
            pl.BlockSpec((D, D), lambda i: (0, 0)),
        ],
        out_specs=[
            pl.BlockSpec((RB, 1), lambda i: (i, 0)),
            pl.BlockSpec((RB, D), lambda i: (i, 0)),
            pl.BlockSpec((RB, D), lambda i: (i, 0)),
        ],
        out_shape=[
            jax.ShapeDtypeStruct((NN, 1), jnp.float32),
            jax.ShapeDtypeStruct((NN, D), jnp.float32),
            jax.ShapeDtypeStruct((NN, D), jnp.float32),
        ],
    )(degp, x, W1)


def _tcd_body(acc_ref, dinv_ref, h_ref, bpre_ref, w_ref, bpost_ref,
              h2_ref, g2_ref):
    dinv = dinv_ref[...]
    z = dinv * (acc_ref[0] + acc_ref[1] + dinv * h_ref[...]) + bpre_ref[...]
    z = jnp.maximum(z, 0.0)
    h2 = (jnp.dot(z, w_ref[...], preferred_element_type=jnp.float32)
          + bpost_ref[...])
    h2_ref[...] = h2
    g2_ref[...] = dinv * h2


def _tcd_call(acc, dinv, h, b_pre, W, b_post):
    return pl.pallas_call(
        _tcd_body,
        grid=(NN // RB,),
        in_specs=[
            pl.BlockSpec((NC, RB, D), lambda i: (0, i, 0)),
            pl.BlockSpec((RB, 1), lambda i: (i, 0)),
            pl.BlockSpec((RB, D), lambda i: (i, 0)),
            pl.BlockSpec((1, D), lambda i: (0, 0)),
            pl.BlockSpec((D, D), lambda i: (0, 0)),
            pl.BlockSpec((1, D), lambda i: (0, 0)),
        ],
        out_specs=[
            pl.BlockSpec((RB, D), lambda i: (i, 0)),
            pl.BlockSpec((RB, D), lambda i: (i, 0)),
        ],
        out_shape=[
            jax.ShapeDtypeStruct((NN, D), jnp.float32),
            jax.ShapeDtypeStruct((NN, D), jnp.float32),
        ],
    )(acc, dinv, h, b_pre, W, b_post)


@jax.jit
def kernel(x, edge_index, W1, b1, W2, b2, Wfc, bfc):
    ncls = Wfc.shape[1]
    ei = edge_index.astype(jnp.int32)
    npad = NEP - NE
    src = jnp.concatenate(
        [ei[0], jnp.zeros((npad,), jnp.int32)]).reshape(NW, NCHUNK, CH)
    dst = jnp.concatenate(
        [ei[1], jnp.full((npad,), TRASH, jnp.int32)]).reshape(NW, NCHUNK, CH)
    ones8 = jnp.ones((CH, DEGW), jnp.float32)
    zeros8 = jnp.zeros((RPT, DEGW), jnp.float32)
    zerosd = jnp.zeros((RPT, D), jnp.float32)

    degp = _deg_call(dst, ones8, zeros8)
    degp = degp[:, :1].reshape(NC, NNP, 1)[:, :NN]
    dinv, h1, g1 = _tcb_call(degp, x, W1)

    wfc_pad = jnp.pad(Wfc, ((0, 0), (0, D - ncls)))
    bfc_pad = jnp.pad(bfc, (0, D - ncls))
    zcol = jnp.zeros((1, D), jnp.float32)

    acc1 = _scatter_call(g1, src, dst, zerosd).reshape(NC, NNP, D)[:, :NN]
    h2, g2 = _tcd_call(acc1, dinv, h1, b1.reshape(1, D), W2, zcol)
    acc2 = _scatter_call(g2, src, dst, zerosd).reshape(NC, NNP, D)[:, :NN]
    hf, _ = _tcd_call(acc2, dinv, h2, b2.reshape(1, D), wfc_pad,
                      bfc_pad.reshape(1, D))
    return hf[:, :ncls]


# speedup vs baseline: 25.0285x; 2.3499x over previous
import functools

import jax
import jax.numpy as jnp
from jax import lax
from jax.experimental import pallas as pl
from jax.experimental.pallas import tpu as pltpu
from jax.experimental.pallas import tpu_sc as plsc

NN = 10000
NE = 320000
D = 128
NC, NS = 2, 16
NW = NC * NS
CH = 128
NCHUNK = 80
EPW = NCHUNK * CH
NEP = NW * EPW
NNP = 10112
RPT = NNP // NS
DEGW = 128
TRASH = NN

_MESH = plsc.VectorSubcoreMesh(core_axis_name="c", subcore_axis_name="s")


def _deg_body(dst_hbm, ones_hbm, zeros_hbm, out_hbm, idx_v, ones_v, acc_sh):
    c = lax.axis_index("c")
    s = lax.axis_index("s")
    w = c * NS + s
    pltpu.sync_copy(zeros_hbm, acc_sh.at[pl.ds(s * RPT, RPT)])
    pltpu.sync_copy(ones_hbm, ones_v)
    pltpu.sync_copy(dst_hbm.at[w], idx_v)
    plsc.subcore_barrier()

    def body(j, carry):
        pltpu.sync_copy(ones_v, acc_sh.at[idx_v.at[j]], add=True)
        return carry

    lax.fori_loop(0, NCHUNK, body, 0)
    plsc.subcore_barrier()
    pltpu.sync_copy(
        acc_sh.at[pl.ds(s * RPT, RPT)],
        out_hbm.at[pl.ds(c * NNP + s * RPT, RPT)],
    )


_deg_call = pl.kernel(
    _deg_body,
    out_type=jax.ShapeDtypeStruct((NC * NNP, DEGW), jnp.float32),
    mesh=_MESH,
    scratch_types=[
        pltpu.VMEM((NCHUNK, CH), jnp.int32),
        pltpu.VMEM((CH, DEGW), jnp.float32),
        pltpu.VMEM_SHARED((NNP, DEGW), jnp.float32),
    ],
)


NPHASE = 2
PH = NCHUNK // NPHASE


def _scat_body(g_hbm, src_hbm, dst_hbm, zeros_hbm, out_hbm,
               src_v, dst_v, rows0, rows1, sem0, sem1, acc_sh):
    c = lax.axis_index("c")
    s = lax.axis_index("s")
    w = c * NS + s
    pltpu.sync_copy(zeros_hbm, acc_sh.at[pl.ds(s * RPT, RPT)])
    plsc.subcore_barrier()

    for p in range(NPHASE):
        pltpu.sync_copy(src_hbm.at[w, pl.ds(p * PH, PH)], src_v)
        pltpu.sync_copy(dst_hbm.at[w, pl.ds(p * PH, PH)], dst_v)
        pltpu.async_copy(g_hbm.at[src_v.at[0]], rows0, sem0)

        def body(t, carry):
            j0 = 2 * t
            pltpu.async_copy(g_hbm.at[src_v.at[j0 + 1]], rows1, sem1)
            pltpu.make_async_copy(g_hbm.at[src_v.at[j0]], rows0, sem0).wait()
            pltpu.sync_copy(rows0, acc_sh.at[dst_v.at[j0]], add=True)

            @pl.when(j0 + 2 < PH)
            def _():
                pltpu.async_copy(g_hbm.at[src_v.at[j0 + 2]], rows0, sem0)

            pltpu.make_async_copy(
                g_hbm.at[src_v.at[j0 + 1]], rows1, sem1).wait()
            pltpu.sync_copy(rows1, acc_sh.at[dst_v.at[j0 + 1]], add=True)
            return carry

        lax.fori_loop(0, PH // 2, body, 0)
    plsc.subcore_barrier()
    pltpu.sync_copy(
        acc_sh.at[pl.ds(s * RPT, RPT)],
        out_hbm.at[pl.ds(c * NNP + s * RPT, RPT)],
    )


_scatter_call = pl.kernel(
    _scat_body,
    out_type=jax.ShapeDtypeStruct((NC * NNP, D), jnp.float32),
    mesh=_MESH,
    scratch_types=[
        pltpu.VMEM((PH, CH), jnp.int32),
        pltpu.VMEM((PH, CH), jnp.int32),
        pltpu.VMEM((CH, D), jnp.float32),
        pltpu.VMEM((CH, D), jnp.float32),
        pltpu.SemaphoreType.DMA,
        pltpu.SemaphoreType.DMA,
        pltpu.VMEM_SHARED((NNP, D), jnp.float32),
    ],
)


RB = 1000


def _tcb_body(degp_ref, x_ref, w_ref, dinv_ref, h_ref, g_ref):
    deg = degp_ref[0] + degp_ref[1] + 1.0
    dinv = lax.rsqrt(deg)
    h = jnp.dot(x_ref[...], w_ref[...], preferred_element_type=jnp.float32)
    dinv_ref[...] = dinv
    h_ref[...] = h
    g_ref[...] = dinv * h


def _tcb_call(degp, x, W1):
    return pl.pallas_call(
        _tcb_body,
        grid=(NN // RB,),
        in_specs=[
            pl.BlockSpec((NC, RB, 1), lambda i: (0, i, 0)),
            pl.BlockSpec((RB, D), lambda i: (i, 0)),
            pl.BlockSpec((D, D), lambda i: (0, 0)),
        ],
        out_specs=[
            pl.BlockSpec((RB, 1), lambda i: (i, 0)),
            pl.BlockSpec((RB, D), lambda i: (i, 0)),
            pl.BlockSpec((RB, D), lambda i: (i, 0)),
        ],
        out_shape=[
            jax.ShapeDtypeStruct((NN, 1), jnp.float32),
            jax.ShapeDtypeStruct((NN, D), jnp.float32),
            jax.ShapeDtypeStruct((NN, D), jnp.float32),
        ],
    )(degp, x, W1)


def _tcd_body(acc_ref, dinv_ref, h_ref, bpre_ref, w_ref, bpost_ref,
              h2_ref, g2_ref):
    dinv = dinv_ref[...]
    z = dinv * (acc_ref[0] + acc_ref[1] + dinv * h_ref[...]) + bpre_ref[...]
    z = jnp.maximum(z, 0.0)
    h2 = (jnp.dot(z, w_ref[...], preferred_element_type=jnp.float32)
          + bpost_ref[...])
    h2_ref[...] = h2
    g2_ref[...] = dinv * h2


def _tcd_call(acc, dinv, h, b_pre, W, b_post):
    return pl.pallas_call(
        _tcd_body,
        grid=(NN // RB,),
        in_specs=[
            pl.BlockSpec((NC, RB, D), lambda i: (0, i, 0)),
            pl.BlockSpec((RB, 1), lambda i: (i, 0)),
            pl.BlockSpec((RB, D), lambda i: (i, 0)),
            pl.BlockSpec((1, D), lambda i: (0, 0)),
            pl.BlockSpec((D, D), lambda i: (0, 0)),
            pl.BlockSpec((1, D), lambda i: (0, 0)),
        ],
        out_specs=[
            pl.BlockSpec((RB, D), lambda i: (i, 0)),
            pl.BlockSpec((RB, D), lambda i: (i, 0)),
        ],
        out_shape=[
            jax.ShapeDtypeStruct((NN, D), jnp.float32),
            jax.ShapeDtypeStruct((NN, D), jnp.float32),
        ],
    )(acc, dinv, h, b_pre, W, b_post)


@jax.jit
def kernel(x, edge_index, W1, b1, W2, b2, Wfc, bfc):
    ncls = Wfc.shape[1]
    ei = edge_index.astype(jnp.int32)
    npad = NEP - NE
    it = jnp.arange(npad, dtype=jnp.int32)
    src = jnp.concatenate(
        [ei[0], (it * 131) % NN]).reshape(NW, NCHUNK, CH)
    dst = jnp.concatenate(
        [ei[1], TRASH + it % (NNP - NN)]).reshape(NW, NCHUNK, CH)
    ones8 = jnp.ones((CH, DEGW), jnp.float32)
    zeros8 = jnp.zeros((RPT, DEGW), jnp.float32)
    zerosd = jnp.zeros((RPT, D), jnp.float32)

    degp = _deg_call(dst, ones8, zeros8)
    degp = degp[:, :1].reshape(NC, NNP, 1)[:, :NN]
    dinv, h1, g1 = _tcb_call(degp, x, W1)

    wfc_pad = jnp.pad(Wfc, ((0, 0), (0, D - ncls)))
    bfc_pad = jnp.pad(bfc, (0, D - ncls))
    zcol = jnp.zeros((1, D), jnp.float32)

    acc1 = _scatter_call(g1, src, dst, zerosd).reshape(NC, NNP, D)[:, :NN]
    h2, g2 = _tcd_call(acc1, dinv, h1, b1.reshape(1, D), W2, zcol)
    acc2 = _scatter_call(g2, src, dst, zerosd).reshape(NC, NNP, D)[:, :NN]
    hf, _ = _tcd_call(acc2, dinv, h2, b2.reshape(1, D), wfc_pad,
                      bfc_pad.reshape(1, D))
    return hf[:, :ncls]


# speedup vs baseline: 26.4910x; 1.0584x over previous
import functools

import jax
import jax.numpy as jnp
from jax import lax
from jax.experimental import pallas as pl
from jax.experimental.pallas import tpu as pltpu
from jax.experimental.pallas import tpu_sc as plsc

NN = 10000
NE = 320000
D = 128
NC, NS = 2, 16
NW = NC * NS
CH = 128
NCHUNK = 80
EPW = NCHUNK * CH
NEP = NW * EPW
NNP = 10112
RPT = NNP // NS
RPT_LAST = RPT - (NNP - NN)
DEGW = 128
TRASH = NN

_MESH = plsc.VectorSubcoreMesh(core_axis_name="c", subcore_axis_name="s")


def _deg_body(dst_hbm, ones_hbm, zeros_hbm, out_hbm, idx_v, ones_v, acc_sh):
    c = lax.axis_index("c")
    s = lax.axis_index("s")
    w = c * NS + s
    pltpu.sync_copy(zeros_hbm, acc_sh.at[pl.ds(s * RPT, RPT)])
    pltpu.sync_copy(ones_hbm, ones_v)
    pltpu.sync_copy(dst_hbm.at[w], idx_v)
    plsc.subcore_barrier()

    def body(j, carry):
        pltpu.sync_copy(ones_v, acc_sh.at[idx_v.at[j]], add=True)
        return carry

    lax.fori_loop(0, NCHUNK, body, 0)
    plsc.subcore_barrier()
    _writeback(acc_sh, out_hbm, c, s)


def _writeback(acc_sh, out_hbm, c, s):
    @pl.when(s < NS - 1)
    def _():
        pltpu.sync_copy(acc_sh.at[pl.ds(s * RPT, RPT)],
                        out_hbm.at[pl.ds(c * NN + s * RPT, RPT)])

    @pl.when(s == NS - 1)
    def _():
        pltpu.sync_copy(acc_sh.at[pl.ds(s * RPT, RPT_LAST)],
                        out_hbm.at[pl.ds(c * NN + s * RPT, RPT_LAST)])


_deg_call = pl.kernel(
    _deg_body,
    out_type=jax.ShapeDtypeStruct((NC * NN, DEGW), jnp.float32),
    mesh=_MESH,
    scratch_types=[
        pltpu.VMEM((NCHUNK, CH), jnp.int32),
        pltpu.VMEM((CH, DEGW), jnp.float32),
        pltpu.VMEM_SHARED((NNP, DEGW), jnp.float32),
    ],
)


NPHASE = 2
PH = NCHUNK // NPHASE


def _scat_body(g_hbm, src_hbm, dst_hbm, zeros_hbm, out_hbm,
               src_v, dst_v, rows0, rows1, sem0, sem1, acc_sh):
    c = lax.axis_index("c")
    s = lax.axis_index("s")
    w = c * NS + s
    pltpu.sync_copy(zeros_hbm, acc_sh.at[pl.ds(s * RPT, RPT)])
    plsc.subcore_barrier()

    for p in range(NPHASE):
        pltpu.sync_copy(src_hbm.at[w, pl.ds(p * PH, PH)], src_v)
        pltpu.sync_copy(dst_hbm.at[w, pl.ds(p * PH, PH)], dst_v)
        pltpu.async_copy(g_hbm.at[src_v.at[0]], rows0, sem0)

        def body(t, carry):
            j0 = 2 * t
            pltpu.async_copy(g_hbm.at[src_v.at[j0 + 1]], rows1, sem1)
            pltpu.make_async_copy(g_hbm.at[src_v.at[j0]], rows0, sem0).wait()
            pltpu.sync_copy(rows0, acc_sh.at[dst_v.at[j0]], add=True)

            @pl.when(j0 + 2 < PH)
            def _():
                pltpu.async_copy(g_hbm.at[src_v.at[j0 + 2]], rows0, sem0)

            pltpu.make_async_copy(
                g_hbm.at[src_v.at[j0 + 1]], rows1, sem1).wait()
            pltpu.sync_copy(rows1, acc_sh.at[dst_v.at[j0 + 1]], add=True)
            return carry

        lax.fori_loop(0, PH // 2, body, 0)
    plsc.subcore_barrier()
    _writeback(acc_sh, out_hbm, c, s)


_scatter_call = pl.kernel(
    _scat_body,
    out_type=jax.ShapeDtypeStruct((NC * NN, D), jnp.float32),
    mesh=_MESH,
    scratch_types=[
        pltpu.VMEM((PH, CH), jnp.int32),
        pltpu.VMEM((PH, CH), jnp.int32),
        pltpu.VMEM((CH, D), jnp.float32),
        pltpu.VMEM((CH, D), jnp.float32),
        pltpu.SemaphoreType.DMA,
        pltpu.SemaphoreType.DMA,
        pltpu.VMEM_SHARED((NNP, D), jnp.float32),
    ],
)


RB = 1000


def _tcb_body(degp_ref, x_ref, w_ref, dinv_ref, h_ref, g_ref):
    deg = degp_ref[0] + degp_ref[1] + 1.0
    dinv = lax.rsqrt(deg)
    h = jnp.dot(x_ref[...], w_ref[...], preferred_element_type=jnp.float32)
    dinv_ref[...] = dinv
    h_ref[...] = h
    g_ref[...] = dinv * h


def _tcb_call(degp, x, W1):
    return pl.pallas_call(
        _tcb_body,
        grid=(NN // RB,),
        in_specs=[
            pl.BlockSpec((NC, RB, 1), lambda i: (0, i, 0)),
            pl.BlockSpec((RB, D), lambda i: (i, 0)),
            pl.BlockSpec((D, D), lambda i: (0, 0)),
        ],
        out_specs=[
            pl.BlockSpec((RB, 1), lambda i: (i, 0)),
            pl.BlockSpec((RB, D), lambda i: (i, 0)),
            pl.BlockSpec((RB, D), lambda i: (i, 0)),
        ],
        out_shape=[
            jax.ShapeDtypeStruct((NN, 1), jnp.float32),
            jax.ShapeDtypeStruct((NN, D), jnp.float32),
            jax.ShapeDtypeStruct((NN, D), jnp.float32),
        ],
    )(degp, x, W1)


def _tcd_body(acc_ref, dinv_ref, h_ref, bpre_ref, w_ref, bpost_ref,
              h2_ref, g2_ref):
    dinv = dinv_ref[...]
    z = dinv * (acc_ref[0] + acc_ref[1] + dinv * h_ref[...]) + bpre_ref[...]
    z = jnp.maximum(z, 0.0)
    h2 = (jnp.dot(z, w_ref[...], preferred_element_type=jnp.float32)
          + bpost_ref[...])
    h2_ref[...] = h2
    g2_ref[...] = dinv * h2


def _tcd_call(acc, dinv, h, b_pre, W, b_post):
    return pl.pallas_call(
        _tcd_body,
        grid=(NN // RB,),
        in_specs=[
            pl.BlockSpec((NC, RB, D), lambda i: (0, i, 0)),
            pl.BlockSpec((RB, 1), lambda i: (i, 0)),
            pl.BlockSpec((RB, D), lambda i: (i, 0)),
            pl.BlockSpec((1, D), lambda i: (0, 0)),
            pl.BlockSpec((D, D), lambda i: (0, 0)),
            pl.BlockSpec((1, D), lambda i: (0, 0)),
        ],
        out_specs=[
            pl.BlockSpec((RB, D), lambda i: (i, 0)),
            pl.BlockSpec((RB, D), lambda i: (i, 0)),
        ],
        out_shape=[
            jax.ShapeDtypeStruct((NN, D), jnp.float32),
            jax.ShapeDtypeStruct((NN, D), jnp.float32),
        ],
    )(acc, dinv, h, b_pre, W, b_post)


@jax.jit
def kernel(x, edge_index, W1, b1, W2, b2, Wfc, bfc):
    ncls = Wfc.shape[1]
    ei = edge_index.astype(jnp.int32)
    npad = NEP - NE
    it = jnp.arange(npad, dtype=jnp.int32)
    src = jnp.concatenate(
        [ei[0], (it * 131) % NN]).reshape(NW, NCHUNK, CH)
    dst = jnp.concatenate(
        [ei[1], TRASH + it % (NNP - NN)]).reshape(NW, NCHUNK, CH)
    ones8 = jnp.ones((CH, DEGW), jnp.float32)
    zeros8 = jnp.zeros((RPT, DEGW), jnp.float32)
    zerosd = jnp.zeros((RPT, D), jnp.float32)

    degp = _deg_call(dst, ones8, zeros8)
    degp = degp[:, :1].reshape(NC, NN, 1)
    dinv, h1, g1 = _tcb_call(degp, x, W1)

    wfc_pad = jnp.pad(Wfc, ((0, 0), (0, D - ncls)))
    bfc_pad = jnp.pad(bfc, (0, D - ncls))
    zcol = jnp.zeros((1, D), jnp.float32)

    acc1 = _scatter_call(g1, src, dst, zerosd).reshape(NC, NN, D)
    h2, g2 = _tcd_call(acc1, dinv, h1, b1.reshape(1, D), W2, zcol)
    acc2 = _scatter_call(g2, src, dst, zerosd).reshape(NC, NN, D)
    hf, _ = _tcd_call(acc2, dinv, h2, b2.reshape(1, D), wfc_pad,
                      bfc_pad.reshape(1, D))
    return hf[:, :ncls]
